# baseline (speedup 1.0000x reference)
import jax
import jax.numpy as jnp
from jax.experimental import pallas as pl

_BT = 1024


def _gate_block(x, wt, bias):
    logits = jnp.dot(x, wt, preferred_element_type=jnp.float32) + bias
    m = jnp.max(logits, axis=1, keepdims=True)
    e = jnp.exp(logits - m)
    s = jnp.sum(e, axis=1, keepdims=True)
    sm = e / s
    v = jnp.max(sm, axis=1, keepdims=True)
    lane = jax.lax.broadcasted_iota(jnp.int32, sm.shape, 1)
    idx = jnp.min(jnp.where(sm >= v, lane, sm.shape[1]), axis=1, keepdims=True)
    return idx, v


def _gate_body(x0_ref, x1_ref, wt_ref, bias_ref,
               idx0_ref, score0_ref, idx1_ref, score1_ref):
    wt = wt_ref[...]
    bias = bias_ref[...]
    i0, v0 = _gate_block(x0_ref[...], wt, bias)
    idx0_ref[...] = i0
    score0_ref[...] = v0
    i1, v1 = _gate_block(x1_ref[...], wt, bias)
    idx1_ref[...] = i1
    score1_ref[...] = v1


def kernel(inp, W, b):
    T, D = inp.shape
    E = W.shape[0]
    wt = W.T
    bias = b.reshape(1, E)
    half = T // 2
    steps = half // _BT
    outs = pl.pallas_call(
        _gate_body,
        grid=(steps,),
        in_specs=[
            pl.BlockSpec((_BT, D), lambda i: (2 * i, 0)),
            pl.BlockSpec((_BT, D), lambda i: (2 * i + 1, 0)),
            pl.BlockSpec((D, E), lambda i: (0, 0)),
            pl.BlockSpec((1, E), lambda i: (0, 0)),
        ],
        out_specs=[
            pl.BlockSpec((_BT, 1), lambda i: (i, 0)),
            pl.BlockSpec((_BT, 1), lambda i: (i, 0)),
            pl.BlockSpec((_BT, 1), lambda i: (i, 0)),
            pl.BlockSpec((_BT, 1), lambda i: (i, 0)),
        ],
        out_shape=[
            jax.ShapeDtypeStruct((half, 1), jnp.int32),
            jax.ShapeDtypeStruct((half, 1), jnp.float32),
            jax.ShapeDtypeStruct((half, 1), jnp.int32),
            jax.ShapeDtypeStruct((half, 1), jnp.float32),
        ],
    )(inp, inp, wt, bias)
    idx0, score0, idx1, score1 = outs
    idx = jnp.stack([idx0.reshape(steps, _BT), idx1.reshape(steps, _BT)],
                    axis=1).reshape(T, 1)
    score = jnp.stack([score0.reshape(steps, _BT), score1.reshape(steps, _BT)],
                      axis=1).reshape(T, 1)
    return (idx.astype(jnp.int64), score)


# speedup vs baseline: 1.0395x; 1.0395x over previous
import jax
import jax.numpy as jnp
from jax.experimental import pallas as pl

_BT = 1024


def _gate_block(x, wt, bias):
    logits = jnp.dot(x, wt, preferred_element_type=jnp.float32) + bias
    m = jnp.max(logits, axis=1, keepdims=True)
    e = jnp.exp(logits - m)
    s = jnp.sum(e, axis=1, keepdims=True)
    sm = e / s
    v = 1.0 / s
    lane = jax.lax.broadcasted_iota(jnp.int32, sm.shape, 1).astype(jnp.float32)
    idxf = jnp.min(jnp.where(sm >= v, lane, float(sm.shape[1])),
                   axis=1, keepdims=True)
    return idxf.astype(jnp.int32), v


def _gate_body(x0_ref, x1_ref, wt_ref, bias_ref,
               idx0_ref, score0_ref, idx1_ref, score1_ref):
    wt = wt_ref[...]
    bias = bias_ref[...]
    i0, v0 = _gate_block(x0_ref[...], wt, bias)
    idx0_ref[...] = i0
    score0_ref[...] = v0
    i1, v1 = _gate_block(x1_ref[...], wt, bias)
    idx1_ref[...] = i1
    score1_ref[...] = v1


def kernel(inp, W, b):
    T, D = inp.shape
    E = W.shape[0]
    wt = W.T
    bias = b.reshape(1, E)
    half = T // 2
    steps = half // _BT
    outs = pl.pallas_call(
        _gate_body,
        grid=(steps,),
        in_specs=[
            pl.BlockSpec((_BT, D), lambda i: (2 * i, 0)),
            pl.BlockSpec((_BT, D), lambda i: (2 * i + 1, 0)),
            pl.BlockSpec((D, E), lambda i: (0, 0)),
            pl.BlockSpec((1, E), lambda i: (0, 0)),
        ],
        out_specs=[
            pl.BlockSpec((_BT, 1), lambda i: (i, 0)),
            pl.BlockSpec((_BT, 1), lambda i: (i, 0)),
            pl.BlockSpec((_BT, 1), lambda i: (i, 0)),
            pl.BlockSpec((_BT, 1), lambda i: (i, 0)),
        ],
        out_shape=[
            jax.ShapeDtypeStruct((half, 1), jnp.int32),
            jax.ShapeDtypeStruct((half, 1), jnp.float32),
            jax.ShapeDtypeStruct((half, 1), jnp.int32),
            jax.ShapeDtypeStruct((half, 1), jnp.float32),
        ],
    )(inp, inp, wt, bias)
    idx0, score0, idx1, score1 = outs
    idx = jnp.stack([idx0.reshape(steps, _BT), idx1.reshape(steps, _BT)],
                    axis=1).reshape(T, 1)
    score = jnp.stack([score0.reshape(steps, _BT), score1.reshape(steps, _BT)],
                      axis=1).reshape(T, 1)
    return (idx.astype(jnp.int64), score)
